# Initial kernel scaffold; baseline (speedup 1.0000x reference)
import jax
import jax.numpy as jnp
from jax.experimental import pallas as pl


def kernel(X, A, W, a):
    raise NotImplementedError("write your pallas kernel here")


# speedup vs baseline: 2.3764x; 2.3764x over previous
import jax
import jax.numpy as jnp
from jax.experimental import pallas as pl


def _project_kernel(x_ref, w_ref, z_ref):
    z_ref[...] = jnp.dot(x_ref[...], w_ref[...],
                         preferred_element_type=jnp.float32)


def _gat_kernel(a_ref, z_ref, zi_ref, adj_ref, o_ref):
    z = z_ref[...]
    zi = zi_ref[...]
    s = jax.lax.dot_general(zi, a_ref[0:1, :], (((1,), (1,)), ((), ())),
                            preferred_element_type=jnp.float32)
    t = jax.lax.dot_general(a_ref[1:2, :], z, (((1,), (1,)), ((), ())),
                            preferred_element_type=jnp.float32)
    e = s + t
    e = jnp.where(e >= 0, e, 0.2 * e)
    e = jnp.where(adj_ref[...] > 0, e, 0.0)
    m = jnp.max(e, axis=1, keepdims=True)
    p = jnp.exp(e - m)
    l = jnp.sum(p, axis=1, keepdims=True)
    o_ref[...] = jnp.dot(p, z, preferred_element_type=jnp.float32) / l


def _pick_block(n, target):
    for b in range(min(target, n), 0, -1):
        if n % b == 0:
            return b
    return n


def kernel(X, A, W, a):
    n, _ = X.shape
    dout = W.shape[1]
    a2r = a.reshape(2, dout).astype(jnp.float32)

    z = pl.pallas_call(
        _project_kernel,
        out_shape=jax.ShapeDtypeStruct((n, dout), jnp.float32),
    )(X, W)

    br = _pick_block(n, 200)
    ni = n // br

    h = pl.pallas_call(
        _gat_kernel,
        grid=(ni,),
        in_specs=[
            pl.BlockSpec((2, dout), lambda i: (0, 0)),
            pl.BlockSpec((n, dout), lambda i: (0, 0)),
            pl.BlockSpec((br, dout), lambda i: (i, 0)),
            pl.BlockSpec((br, n), lambda i: (i, 0)),
        ],
        out_specs=pl.BlockSpec((br, dout), lambda i: (i, 0)),
        out_shape=jax.ShapeDtypeStruct((n, dout), jnp.float32),
    )(a2r, z, z, A)
    return h


# speedup vs baseline: 2.4523x; 1.0319x over previous
import jax
import jax.numpy as jnp
from jax.experimental import pallas as pl


def _project_kernel(x_ref, w_ref, a_ref, z_ref, s_ref, t_ref):
    z = jnp.dot(x_ref[...], w_ref[...], preferred_element_type=jnp.float32)
    z_ref[...] = z
    s_ref[...] = jax.lax.dot_general(z, a_ref[0:1, :], (((1,), (1,)), ((), ())),
                                     preferred_element_type=jnp.float32)
    t_ref[...] = jax.lax.dot_general(a_ref[1:2, :], z, (((1,), (1,)), ((), ())),
                                     preferred_element_type=jnp.float32)


def _gat_kernel(s_ref, t_ref, z_ref, adj_ref, o_ref):
    z = z_ref[...]
    e = s_ref[...] + t_ref[...]
    e = jnp.where(e >= 0, e, 0.2 * e)
    e = jnp.where(adj_ref[...] > 0, e, 0.0)
    m = jnp.max(e, axis=1, keepdims=True)
    p = jnp.exp(e - m)
    l = jnp.sum(p, axis=1, keepdims=True)
    o_ref[...] = jnp.dot(p, z, preferred_element_type=jnp.float32) / l


def _pick_block(n, target):
    for b in range(min(target, n), 0, -1):
        if n % b == 0:
            return b
    return n


def kernel(X, A, W, a):
    n, _ = X.shape
    dout = W.shape[1]
    a2r = a.reshape(2, dout).astype(jnp.float32)

    z, s, t = pl.pallas_call(
        _project_kernel,
        out_shape=[
            jax.ShapeDtypeStruct((n, dout), jnp.float32),
            jax.ShapeDtypeStruct((n, 1), jnp.float32),
            jax.ShapeDtypeStruct((1, n), jnp.float32),
        ],
    )(X, W, a2r)

    br = _pick_block(n, 200)
    ni = n // br

    h = pl.pallas_call(
        _gat_kernel,
        grid=(ni,),
        in_specs=[
            pl.BlockSpec((br, 1), lambda i: (i, 0)),
            pl.BlockSpec((1, n), lambda i: (0, 0)),
            pl.BlockSpec((n, dout), lambda i: (0, 0)),
            pl.BlockSpec((br, n), lambda i: (i, 0)),
        ],
        out_specs=pl.BlockSpec((br, dout), lambda i: (i, 0)),
        out_shape=jax.ShapeDtypeStruct((n, dout), jnp.float32),
    )(s, t, z, A)
    return h


# speedup vs baseline: 2.4609x; 1.0035x over previous
import jax
import jax.numpy as jnp
from jax.experimental import pallas as pl


_LOG2E = 1.4426950408889634


def _project_kernel(x_ref, w_ref, a_ref, zb_ref, s_ref, t_ref):
    z = jnp.dot(x_ref[...], w_ref[...], preferred_element_type=jnp.float32)
    zb_ref[...] = z.astype(jnp.bfloat16)
    s_ref[...] = _LOG2E * jax.lax.dot_general(
        z, a_ref[0:1, :], (((1,), (1,)), ((), ())),
        preferred_element_type=jnp.float32)
    t_ref[...] = _LOG2E * jax.lax.dot_general(
        a_ref[1:2, :], z, (((1,), (1,)), ((), ())),
        preferred_element_type=jnp.float32)


def _gat_kernel(s_ref, t_ref, zb_ref, adj_ref, o_ref):
    x = s_ref[...] + t_ref[...]
    e = jnp.maximum(x, 0.2 * x)
    e = jnp.where(adj_ref[...] > 0, e, 0.0)
    m = jnp.max(e, axis=1, keepdims=True)
    p = jnp.exp2(e - m)
    l = jnp.sum(p, axis=1, keepdims=True)
    num = jnp.dot(p.astype(jnp.bfloat16), zb_ref[...],
                  preferred_element_type=jnp.float32)
    o_ref[...] = num / l


def _pick_block(n, target):
    for b in range(min(target, n), 0, -1):
        if n % b == 0:
            return b
    return n


def kernel(X, A, W, a):
    n, _ = X.shape
    dout = W.shape[1]
    a2r = a.reshape(2, dout).astype(jnp.float32)

    zb, s, t = pl.pallas_call(
        _project_kernel,
        out_shape=[
            jax.ShapeDtypeStruct((n, dout), jnp.bfloat16),
            jax.ShapeDtypeStruct((n, 1), jnp.float32),
            jax.ShapeDtypeStruct((1, n), jnp.float32),
        ],
    )(X, W, a2r)

    br = _pick_block(n, 200)
    ni = n // br

    h = pl.pallas_call(
        _gat_kernel,
        grid=(ni,),
        in_specs=[
            pl.BlockSpec((br, 1), lambda i: (i, 0)),
            pl.BlockSpec((1, n), lambda i: (0, 0)),
            pl.BlockSpec((n, dout), lambda i: (0, 0)),
            pl.BlockSpec((br, n), lambda i: (i, 0)),
        ],
        out_specs=pl.BlockSpec((br, dout), lambda i: (i, 0)),
        out_shape=jax.ShapeDtypeStruct((n, dout), jnp.float32),
    )(s, t, zb, A)
    return h


# speedup vs baseline: 2.5175x; 1.0230x over previous
import jax
import jax.numpy as jnp
from jax.experimental import pallas as pl


_LOG2E = 1.4426950408889634


def _project_kernel(x_ref, w_ref, a_ref, zb_ref, s_ref, t_ref):
    z = jnp.dot(x_ref[...], w_ref[...], preferred_element_type=jnp.float32)
    zb_ref[...] = z.astype(jnp.bfloat16)
    s_ref[...] = _LOG2E * jax.lax.dot_general(
        z, a_ref[0:1, :], (((1,), (1,)), ((), ())),
        preferred_element_type=jnp.float32)
    t_ref[...] = _LOG2E * jax.lax.dot_general(
        a_ref[1:2, :], z, (((1,), (1,)), ((), ())),
        preferred_element_type=jnp.float32)


def _gat_kernel(s_ref, t_ref, zb_ref, adj_ref, o_ref):
    x = s_ref[...] + t_ref[...]
    e = jnp.maximum(x, 0.2 * x)
    e = jnp.where(adj_ref[...] > 0, e, 0.0)
    m = jnp.max(e, axis=1, keepdims=True)
    p = jnp.exp2(e - m)
    l = jnp.sum(p, axis=1, keepdims=True)
    num = jnp.dot(p.astype(jnp.bfloat16), zb_ref[...],
                  preferred_element_type=jnp.float32)
    o_ref[...] = num / l


def _pick_block(n, target):
    for b in range(min(target, n), 0, -1):
        if n % b == 0:
            return b
    return n


def kernel(X, A, W, a):
    n, _ = X.shape
    dout = W.shape[1]
    a2r = a.reshape(2, dout).astype(jnp.float32)

    zb, s, t = pl.pallas_call(
        _project_kernel,
        out_shape=[
            jax.ShapeDtypeStruct((n, dout), jnp.bfloat16),
            jax.ShapeDtypeStruct((n, 1), jnp.float32),
            jax.ShapeDtypeStruct((1, n), jnp.float32),
        ],
    )(X, W, a2r)

    br = _pick_block(n, 400)
    ni = n // br

    h = pl.pallas_call(
        _gat_kernel,
        grid=(ni,),
        in_specs=[
            pl.BlockSpec((br, 1), lambda i: (i, 0)),
            pl.BlockSpec((1, n), lambda i: (0, 0)),
            pl.BlockSpec((n, dout), lambda i: (0, 0)),
            pl.BlockSpec((br, n), lambda i: (i, 0)),
        ],
        out_specs=pl.BlockSpec((br, dout), lambda i: (i, 0)),
        out_shape=jax.ShapeDtypeStruct((n, dout), jnp.float32),
    )(s, t, zb, A)
    return h


# speedup vs baseline: 2.5661x; 1.0193x over previous
import functools

import jax
import jax.numpy as jnp
from jax.experimental import pallas as pl


_LOG2E = 1.4426950408889634


def _project_kernel(x_ref, w_ref, a_ref, zb_ref, s_ref, t_ref):
    z = jnp.dot(x_ref[...], w_ref[...], preferred_element_type=jnp.float32)
    zb_ref[...] = z.astype(jnp.bfloat16)
    s_ref[...] = _LOG2E * jax.lax.dot_general(
        z, a_ref[0:1, :], (((1,), (1,)), ((), ())),
        preferred_element_type=jnp.float32)
    t_ref[...] = _LOG2E * jax.lax.dot_general(
        a_ref[1:2, :], z, (((1,), (1,)), ((), ())),
        preferred_element_type=jnp.float32)


def _half(s, t, zb, adj):
    x = s + t
    e = jnp.maximum(x, 0.2 * x)
    e = jnp.where(adj > 0, e, 0.0)
    m = jnp.max(e, axis=1, keepdims=True)
    p = jnp.exp2(e - m)
    l = jnp.sum(p, axis=1, keepdims=True)
    num = jnp.dot(p.astype(jnp.bfloat16), zb,
                  preferred_element_type=jnp.float32)
    return num / l


def _gat_kernel(br, s_ref, t_ref, zb_ref, adj0_ref, adj1_ref, o_ref):
    t = t_ref[...]
    zb = zb_ref[...]
    o_ref[0:br, :] = _half(s_ref[0:br, :], t, zb, adj0_ref[...])
    o_ref[br:2 * br, :] = _half(s_ref[br:2 * br, :], t, zb, adj1_ref[...])


def _pick_block(n, target):
    for b in range(min(target, n), 0, -1):
        if n % b == 0:
            return b
    return n


def kernel(X, A, W, a):
    n, _ = X.shape
    dout = W.shape[1]
    a2r = a.reshape(2, dout).astype(jnp.float32)

    zb, s, t = pl.pallas_call(
        _project_kernel,
        out_shape=[
            jax.ShapeDtypeStruct((n, dout), jnp.bfloat16),
            jax.ShapeDtypeStruct((n, 1), jnp.float32),
            jax.ShapeDtypeStruct((1, n), jnp.float32),
        ],
    )(X, W, a2r)

    br = _pick_block(n // 2, 200)
    ni = n // (2 * br)

    h = pl.pallas_call(
        functools.partial(_gat_kernel, br),
        grid=(ni,),
        in_specs=[
            pl.BlockSpec((2 * br, 1), lambda i: (i, 0)),
            pl.BlockSpec((1, n), lambda i: (0, 0)),
            pl.BlockSpec((n, dout), lambda i: (0, 0)),
            pl.BlockSpec((br, n), lambda i: (2 * i, 0)),
            pl.BlockSpec((br, n), lambda i: (2 * i + 1, 0)),
        ],
        out_specs=pl.BlockSpec((2 * br, dout), lambda i: (i, 0)),
        out_shape=jax.ShapeDtypeStruct((n, dout), jnp.float32),
    )(s, t, zb, A, A)
    return h
